# baseline (speedup 1.0000x reference)
import functools

import jax
import jax.numpy as jnp
from jax import lax
from jax.experimental import pallas as pl
from jax.experimental.pallas import tpu as pltpu
from jax.experimental.pallas import tpu_sc as plsc

_HIDDEN = 64
_LN2 = 0.6931471805599453
_SQRT2 = 1.4142135623730951


def _log1p_abs(x):
    a = jnp.abs(x)
    y = 1.0 + a
    bits = lax.bitcast_convert_type(y, jnp.int32)
    e = lax.shift_right_logical(bits, 23) - 127
    m_bits = lax.bitwise_or(lax.bitwise_and(bits, 0x007FFFFF), 0x3F800000)
    m = lax.bitcast_convert_type(m_bits, jnp.float32)
    big = m > _SQRT2
    m = jnp.where(big, m * 0.5, m)
    e = jnp.where(big, e + 1, e)
    r = m - 1.0
    p = jnp.float32(1.0 / 9.0)
    for k in range(8, 0, -1):
        c = jnp.float32((1.0 if k % 2 == 1 else -1.0) / k)
        p = p * r + c
    p = p * r
    return e.astype(jnp.float32) * jnp.float32(_LN2) + p


def _scale_slices(seq):
    offs = list(range(0, seq - 15, 16))
    if offs[-1] + 16 < seq:
        offs.append(seq - 16)
    return offs


def kernel(input_ids, num_values, table, num_encoding_vector):
    bsz, seq = input_ids.shape
    nw = 32
    rows_b = bsz // nw
    rb = 4
    nch = rows_b // rb
    half = (seq // 2 + 7) // 8 * 8

    mesh = plsc.VectorSubcoreMesh(core_axis_name="c", subcore_axis_name="s")

    @functools.partial(
        pl.kernel,
        mesh=mesh,
        out_type=jax.ShapeDtypeStruct((bsz, seq, _HIDDEN), jnp.float32),
        compiler_params=pltpu.CompilerParams(
            use_tc_tiling_on_sc=False, needs_layout_passes=False),
        scratch_types=[
            pltpu.VMEM((rows_b, seq), jnp.int32),
            pltpu.VMEM((2, rb, seq), jnp.float32),
            pltpu.VMEM((2, rb, seq, _HIDDEN), jnp.float32),
            pltpu.VMEM((_HIDDEN,), jnp.float32),
            pltpu.SemaphoreType.DMA,
            pltpu.SemaphoreType.DMA,
        ],
    )
    def k(ids_hbm, vals_hbm, table_hbm, nev_hbm, out_hbm,
          idx_v, scale_v, rows_v, nev_v, sem0, sem1):
        sems = (sem0, sem1)
        wid = lax.axis_index("s") * 2 + lax.axis_index("c")
        b0 = wid * rows_b

        pltpu.sync_copy(nev_hbm, nev_v)
        pltpu.sync_copy(ids_hbm.at[pl.ds(b0, rows_b)], idx_v)

        def fire_gathers(c, b):
            descs = []
            for r in range(rb):
                bi = c * rb + r
                for (o, w) in ((0, half), (half, seq - half)):
                    descs.append(pltpu.async_copy(
                        table_hbm.at[idx_v.at[bi, pl.ds(o, w)]],
                        rows_v.at[b, r, pl.ds(o, w)],
                        sems[b],
                    ))
            return descs

        descs = [None, None]
        descs[0] = fire_gathers(0, 0)

        for c in range(nch):
            b = c % 2
            if c + 1 < nch:
                descs[1 - b] = fire_gathers(c + 1, 1 - b)

            pltpu.sync_copy(
                vals_hbm.at[pl.ds(b0 + c * rb, rb)], scale_v.at[b])
            for r in range(rb):
                offs = _scale_slices(seq)
                vs = [scale_v[b, r, pl.ds(t, 16)] for t in offs]
                for t, v in zip(offs, vs):
                    scale_v[b, r, pl.ds(t, 16)] = jnp.sign(v) * _log1p_abs(v)

            for d in descs[b]:
                d.wait()

            nev_regs = [nev_v[pl.ds(kk * 16, 16)] for kk in range(4)]

            for r in range(rb):
                def row_body(j, _, b=b, r=r):
                    s = plsc.load_gather(
                        scale_v.at[b, r], [jnp.full((16,), j, jnp.int32)])
                    for kk in range(4):
                        sl = pl.ds(kk * 16, 16)
                        rows_v[b, r, j, sl] = (
                            rows_v[b, r, j, sl] + s * nev_regs[kk])
                    return 0

                lax.fori_loop(0, seq, row_body, 0)

            pltpu.sync_copy(
                rows_v.at[b], out_hbm.at[pl.ds(b0 + c * rb, rb)])

    return k(input_ids, num_values, table, num_encoding_vector)


# speedup vs baseline: 1.0872x; 1.0872x over previous
import functools

import jax
import jax.numpy as jnp
from jax import lax
from jax.experimental import pallas as pl
from jax.experimental.pallas import tpu as pltpu
from jax.experimental.pallas import tpu_sc as plsc

_HIDDEN = 64
_LN2 = 0.6931471805599453
_SQRT2 = 1.4142135623730951


def _log1p_abs(x):
    a = jnp.abs(x)
    y = 1.0 + a
    bits = lax.bitcast_convert_type(y, jnp.int32)
    e = lax.shift_right_logical(bits, 23) - 127
    m_bits = lax.bitwise_or(lax.bitwise_and(bits, 0x007FFFFF), 0x3F800000)
    m = lax.bitcast_convert_type(m_bits, jnp.float32)
    big = m > _SQRT2
    m = jnp.where(big, m * 0.5, m)
    e = jnp.where(big, e + 1, e)
    r = m - 1.0
    p = jnp.float32(1.0 / 9.0)
    for k in range(8, 0, -1):
        c = jnp.float32((1.0 if k % 2 == 1 else -1.0) / k)
        p = p * r + c
    p = p * r
    return e.astype(jnp.float32) * jnp.float32(_LN2) + p


def _scale_slices(seq):
    offs = list(range(0, seq - 15, 16))
    if offs[-1] + 16 < seq:
        offs.append(seq - 16)
    return offs


def kernel(input_ids, num_values, table, num_encoding_vector):
    bsz, seq = input_ids.shape
    nw = 32
    rows_b = bsz // nw
    rb = 4
    nch = rows_b // rb
    half = (seq // 2 + 7) // 8 * 8

    mesh = plsc.VectorSubcoreMesh(core_axis_name="c", subcore_axis_name="s")

    @functools.partial(
        pl.kernel,
        mesh=mesh,
        out_type=jax.ShapeDtypeStruct((bsz, seq, _HIDDEN), jnp.float32),
        compiler_params=pltpu.CompilerParams(
            use_tc_tiling_on_sc=False, needs_layout_passes=False),
        scratch_types=[
            pltpu.VMEM((rows_b, seq), jnp.int32),
            pltpu.VMEM((2, rb, seq), jnp.float32),
            pltpu.VMEM((2, rb, seq, _HIDDEN), jnp.float32),
            pltpu.VMEM((_HIDDEN,), jnp.float32),
            pltpu.SemaphoreType.DMA,
            pltpu.SemaphoreType.DMA,
        ],
    )
    def k(ids_hbm, vals_hbm, table_hbm, nev_hbm, out_hbm,
          idx_v, scale_v, rows_v, nev_v, sem0, sem1):
        sems = (sem0, sem1)
        wid = lax.axis_index("s") * 2 + lax.axis_index("c")
        b0 = wid * rows_b

        pltpu.sync_copy(nev_hbm, nev_v)
        pltpu.sync_copy(ids_hbm.at[pl.ds(b0, rows_b)], idx_v)

        def fire_gathers(c, b):
            descs = []
            for r in range(rb):
                bi = c * rb + r
                for (o, w) in ((0, half), (half, seq - half)):
                    descs.append(pltpu.async_copy(
                        table_hbm.at[idx_v.at[bi, pl.ds(o, w)]],
                        rows_v.at[b, r, pl.ds(o, w)],
                        sems[b],
                    ))
            return descs

        descs = [None, None]
        descs[0] = fire_gathers(0, 0)

        for c in range(nch):
            b = c % 2
            if c + 1 < nch:
                descs[1 - b] = fire_gathers(c + 1, 1 - b)

            pltpu.sync_copy(
                vals_hbm.at[pl.ds(b0 + c * rb, rb)], scale_v.at[b])
            for r in range(rb):
                offs = _scale_slices(seq)
                vs = [scale_v[b, r, pl.ds(t, 16)] for t in offs]
                for t, v in zip(offs, vs):
                    scale_v[b, r, pl.ds(t, 16)] = jnp.sign(v) * _log1p_abs(v)

            for d in descs[b]:
                d.wait()

            nev_regs = [nev_v[pl.ds(kk * 16, 16)] for kk in range(4)]

            for r in range(rb):
                def row_body(j, _, b=b, r=r):
                    s = plsc.load_gather(
                        scale_v.at[b, r], [jnp.full((16,), j, jnp.int32)])
                    for kk in range(4):
                        sl = pl.ds(kk * 16, 16)
                        rows_v[b, r, j, sl] = (
                            rows_v[b, r, j, sl] + s * nev_regs[kk])
                    return 0

                lax.fori_loop(0, seq, row_body, 0)

            pltpu.sync_copy(
                rows_v.at[b], out_hbm.at[pl.ds(b0 + c * rb, rb)])

    tpad = jnp.pad(table, ((0, (-table.shape[0]) % 8), (0, _HIDDEN)))
    tview = tpad.reshape(tpad.shape[0] * 2, _HIDDEN)
    return k(input_ids * 2, num_values, tview, num_encoding_vector)
